# baseline (speedup 1.0000x reference)
import functools

import jax
import jax.numpy as jnp
from jax import lax
from jax.experimental import pallas as pl
from jax.experimental.pallas import tpu as pltpu
from jax.experimental.pallas import tpu_sc as plsc

BS = 32
QN = 32
FD = 256
NROW = BS * QN
NSEL = 4
LANES = 16
NEG = (BS - 1) * NSEL
LOGN = 128
NEG_FILL = -1e30


def _f16_roundtrip(x):
    c = jnp.float32(8193.0)
    m = jnp.float32(0.75)
    y = x * c
    hi = y - (y - x)
    lo = (x + m) - m
    return jnp.where(jnp.abs(x) >= jnp.float32(2.0 ** -14), hi, lo)


def _prep_body(v_ref, l_ref, g_ref, sim_ref, diff_ref):
    V = v_ref[...]
    L = l_ref[...]
    n2 = jnp.sum(V * V, axis=1, keepdims=True)
    nrm = jnp.maximum(jnp.sqrt(n2), 1e-12)
    Uh = _f16_roundtrip(V / nrm)
    G = lax.dot_general(Uh, Uh, (((1,), (1,)), ((), ())),
                        preferred_element_type=jnp.float32)
    c = jnp.float32(8193.0)
    y = G * c
    g_ref[...] = y - (y - G)
    sim = lax.dot_general(L, V, (((1,), (1,)), ((), ())),
                          preferred_element_type=jnp.float32)
    sim_ref[...] = sim
    sim3 = sim.reshape(BS, BS, QN)
    mn = jnp.min(sim3, axis=2, keepdims=True)
    mx = jnp.max(sim3, axis=2, keepdims=True)
    diff_ref[...] = ((sim3 - mn) / (mx - mn)).reshape(BS, NROW)


def _prep(V, L):
    return pl.pallas_call(
        _prep_body,
        out_shape=[
            jax.ShapeDtypeStruct((NROW, NROW), jnp.float32),
            jax.ShapeDtypeStruct((BS, NROW), jnp.float32),
            jax.ShapeDtypeStruct((BS, NROW), jnp.float32),
        ],
    )(V, L)


def _sc_mine(G, sim, diff):
    info = plsc.get_sparse_core_info()
    nc = info.num_cores

    mesh = plsc.VectorSubcoreMesh(core_axis_name="c", subcore_axis_name="s")

    @functools.partial(
        pl.kernel,
        mesh=mesh,
        compiler_params=pltpu.CompilerParams(needs_layout_passes=False),
        out_type=jax.ShapeDtypeStruct((BS, LOGN), jnp.float32),
        scratch_types=[
            pltpu.VMEM((NROW,), jnp.float32),
            pltpu.VMEM((NROW,), jnp.float32),
            pltpu.VMEM((NROW,), jnp.float32),
            pltpu.VMEM((NROW,), jnp.float32),
            pltpu.VMEM((BS,), jnp.int32),
            pltpu.VMEM((NSEL, BS), jnp.int32),
            pltpu.VMEM((BS, NROW), jnp.float32),
            pltpu.VMEM((LOGN,), jnp.float32),
            pltpu.SemaphoreType.DMA,
        ],
    )
    def body(g_hbm, sim_hbm, diff_hbm, out_hbm, sim_v, diff_v,
             uniq_v, score_v, selidx, selhist, gbuf, logits_v, sem):
        b = lax.axis_index("s") * nc + lax.axis_index("c")
        iota = lax.iota(jnp.int32, LANES)

        pltpu.sync_copy(sim_hbm.at[b], sim_v)
        pltpu.sync_copy(diff_hbm.at[b], diff_v)
        pltpu.sync_copy(diff_hbm.at[b], score_v)

        for it in range(NSEL):
            def sel_a(a, carry):
                sv0, sv1, first = carry
                base = a * QN
                s0 = score_v[pl.ds(base, LANES)]
                s1 = score_v[pl.ds(base + LANES, LANES)]
                m = jnp.maximum(jnp.max(s0), jnp.max(s1))
                big = jnp.int32(9999)
                q0 = jnp.min(jnp.where(s0 == m, iota, big))
                q1 = jnp.min(jnp.where(s1 == m, iota + LANES, big))
                sel = base + jnp.minimum(q0, q1)
                upd = a != b
                sv0 = jnp.where(jnp.logical_and(iota == a, upd), sel, sv0)
                sv1 = jnp.where(jnp.logical_and(iota == a - LANES, upd),
                                sel, sv1)
                first = jnp.where(jnp.logical_and(upd, first < 0), sel, first)
                return sv0, sv1, first

            z16 = jnp.zeros((LANES,), jnp.int32)
            sv0, sv1, first = lax.fori_loop(0, BS, sel_a,
                                            (z16, z16, jnp.int32(-1)))
            sv0 = jnp.where(iota == b, first, sv0)
            sv1 = jnp.where(iota == b - LANES, first, sv1)
            selidx[pl.ds(0, LANES)] = sv0
            selidx[pl.ds(LANES, LANES)] = sv1
            selhist[it, pl.ds(0, LANES)] = sv0
            selhist[it, pl.ds(LANES, LANES)] = sv1

            pltpu.async_copy(g_hbm.at[selidx], gbuf, sem).wait()

            first_round = it == 0

            def upd_v(v, carry):
                sl = pl.ds(v * LANES, LANES)
                ms = [jnp.maximum(gbuf[2 * j, sl], gbuf[2 * j + 1, sl])
                      for j in range(BS // 2)]
                while len(ms) > 1:
                    ms = [jnp.maximum(ms[2 * j], ms[2 * j + 1])
                          for j in range(len(ms) // 2)]
                qv = (1.0 - ms[0]) * 0.5
                u = qv if first_round else jnp.minimum(uniq_v[sl], qv)
                uniq_v[sl] = u
                score_v[sl] = u * diff_v[sl]
                return carry

            lax.fori_loop(0, NROW // LANES, upd_v, 0)

        fill = jnp.full((LANES,), NEG_FILL, jnp.float32)
        for c in range(LOGN // LANES):
            logits_v[pl.ds(c * LANES, LANES)] = fill

        for it in range(NSEL):
            for h in range(2):
                nvec = iota + h * LANES
                live = nvec < BS - 1
                avec = jnp.minimum(nvec + (nvec >= b).astype(jnp.int32),
                                   jnp.int32(BS - 1))
                rows = plsc.load_gather(
                    selhist, [jnp.full((LANES,), it, jnp.int32), avec],
                    mask=live)
                rows = jnp.where(live, rows, 0)
                vals = plsc.load_gather(sim_v, [rows], mask=live)
                posn = jnp.where(live, 1 + nvec * NSEL + it, 0)
                plsc.store_scatter(logits_v, [posn], vals, mask=live)

        p0 = sim_v[pl.ds(b * QN, LANES)]
        p1 = sim_v[pl.ds(b * QN + LANES, LANES)]
        pos = jnp.maximum(jnp.max(p0), jnp.max(p1))
        l0 = logits_v[pl.ds(0, LANES)]
        logits_v[pl.ds(0, LANES)] = jnp.where(iota == 0, pos, l0)

        pltpu.sync_copy(logits_v, out_hbm.at[b])

    return body(G, sim, diff)


def _loss_body(lg_ref, out_ref):
    lg = lg_ref[...]
    m = jnp.max(lg, axis=1, keepdims=True)
    s = jnp.sum(jnp.exp(lg - m), axis=1, keepdims=True)
    lse = m + jnp.log(s)
    logp0 = lg[:, 0:1] - lse
    out_ref[...] = jnp.full((1, 1), -jnp.mean(logp0), jnp.float32)


def _loss(logits):
    return pl.pallas_call(
        _loss_body,
        out_shape=jax.ShapeDtypeStruct((1, 1), jnp.float32),
    )(logits)


def kernel(vis_fs, lan_fs):
    V = vis_fs.reshape(NROW, FD)
    L = lan_fs.reshape(BS, FD)
    G, sim, diff = _prep(V, L)
    logits = _sc_mine(G, sim, diff)
    return _loss(logits).reshape(())


# speedup vs baseline: 1.0789x; 1.0789x over previous
import functools

import jax
import jax.numpy as jnp
from jax import lax
from jax.experimental import pallas as pl
from jax.experimental.pallas import tpu as pltpu
from jax.experimental.pallas import tpu_sc as plsc

BS = 32
QN = 32
FD = 256
NROW = BS * QN
NSEL = 4
LANES = 16
LOGN = 128
NEG_FILL = -1e30
BIG = 9999


def _f16_roundtrip(x):
    c = jnp.float32(8193.0)
    m = jnp.float32(0.75)
    y = x * c
    hi = y - (y - x)
    lo = (x + m) - m
    return jnp.where(jnp.abs(x) >= jnp.float32(2.0 ** -14), hi, lo)


def _prep_body(v_ref, l_ref, g_ref, sim_ref, diff_ref, sel0_ref):
    V = v_ref[...]
    L = l_ref[...]
    n2 = jnp.sum(V * V, axis=1, keepdims=True)
    nrm = jnp.maximum(jnp.sqrt(n2), 1e-12)
    Uh = _f16_roundtrip(V / nrm)
    G = lax.dot_general(Uh, Uh, (((1,), (1,)), ((), ())),
                        preferred_element_type=jnp.float32)
    c = jnp.float32(8193.0)
    y = G * c
    g_ref[...] = y - (y - G)
    sim = lax.dot_general(L, V, (((1,), (1,)), ((), ())),
                          preferred_element_type=jnp.float32)
    sim_ref[...] = sim
    sim3 = sim.reshape(BS, BS, QN)
    mn = jnp.min(sim3, axis=2, keepdims=True)
    mx = jnp.max(sim3, axis=2, keepdims=True)
    diff_ref[...] = ((sim3 - mn) / (mx - mn)).reshape(BS, NROW)
    qio = lax.broadcasted_iota(jnp.int32, (BS, BS, QN), 2)
    qmin = jnp.min(jnp.where(sim3 == mx, qio, BIG), axis=2)
    sel0_ref[...] = qmin + lax.broadcasted_iota(jnp.int32, (BS, BS), 1) * QN


def _prep(V, L):
    return pl.pallas_call(
        _prep_body,
        out_shape=[
            jax.ShapeDtypeStruct((NROW, NROW), jnp.float32),
            jax.ShapeDtypeStruct((BS, NROW), jnp.float32),
            jax.ShapeDtypeStruct((BS, NROW), jnp.float32),
            jax.ShapeDtypeStruct((BS, BS), jnp.int32),
        ],
    )(V, L)


def _sc_mine(G, sim, diff, sel0):
    info = plsc.get_sparse_core_info()
    nc = info.num_cores

    mesh = plsc.VectorSubcoreMesh(core_axis_name="c", subcore_axis_name="s")

    @functools.partial(
        pl.kernel,
        mesh=mesh,
        compiler_params=pltpu.CompilerParams(needs_layout_passes=False),
        out_type=jax.ShapeDtypeStruct((BS, LOGN), jnp.float32),
        scratch_types=[
            pltpu.VMEM((NROW,), jnp.float32),
            pltpu.VMEM((NROW,), jnp.float32),
            pltpu.VMEM((NROW,), jnp.float32),
            pltpu.VMEM((BS,), jnp.int32),
            pltpu.VMEM((NSEL, BS), jnp.int32),
            pltpu.VMEM((BS, NROW), jnp.float32),
            pltpu.VMEM((LOGN,), jnp.float32),
            pltpu.SemaphoreType.DMA,
        ],
    )
    def body(g_hbm, sim_hbm, diff_hbm, sel0_hbm, out_hbm, sim_v, diff_v,
             uniq_v, selidx, selhist, gbuf, logits_v, sem):
        b = lax.axis_index("s") * nc + lax.axis_index("c")
        iota = lax.iota(jnp.int32, LANES)

        pltpu.sync_copy(sim_hbm.at[b], sim_v)
        pltpu.sync_copy(diff_hbm.at[b], diff_v)
        pltpu.sync_copy(sel0_hbm.at[b], selidx)

        def fix_lane_b(sv0, sv1, repl):
            sv0 = jnp.where(iota == b, repl, sv0)
            sv1 = jnp.where(iota == b - LANES, repl, sv1)
            return sv0, sv1

        dupe = plsc.load_gather(selidx, [iota * 0 + (b + 1) % BS])
        sv0, sv1 = fix_lane_b(selidx[pl.ds(0, LANES)],
                              selidx[pl.ds(LANES, LANES)], dupe)
        selidx[pl.ds(0, LANES)] = sv0
        selidx[pl.ds(LANES, LANES)] = sv1
        selhist[0, pl.ds(0, LANES)] = sv0
        selhist[0, pl.ds(LANES, LANES)] = sv1

        copy = pltpu.async_copy(g_hbm.at[selidx], gbuf, sem)

        for it in range(NSEL - 1):
            copy.wait()

            def seg_a(a, carry):
                sv0, sv1, first = carry
                base = a * QN
                subs = []
                for half in range(2):
                    sl = pl.ds(base + half * LANES, LANES)
                    ms = [jnp.maximum(gbuf[2 * j, sl], gbuf[2 * j + 1, sl])
                          for j in range(BS // 2)]
                    while len(ms) > 1:
                        ms = [jnp.maximum(ms[2 * j], ms[2 * j + 1])
                              for j in range(len(ms) // 2)]
                    qv = (1.0 - ms[0]) * 0.5
                    u = qv if it == 0 else jnp.minimum(uniq_v[sl], qv)
                    uniq_v[sl] = u
                    subs.append(u * diff_v[sl])
                s0, s1 = subs
                m = jnp.maximum(jnp.max(s0), jnp.max(s1))
                q0 = jnp.min(jnp.where(s0 == m, iota, jnp.int32(BIG)))
                q1 = jnp.min(jnp.where(s1 == m, iota + LANES, jnp.int32(BIG)))
                sel = base + jnp.minimum(q0, q1)
                upd = a != b
                sv0 = jnp.where(jnp.logical_and(iota == a, upd), sel, sv0)
                sv1 = jnp.where(jnp.logical_and(iota == a - LANES, upd),
                                sel, sv1)
                first = jnp.where(jnp.logical_and(upd, first < 0), sel, first)
                return sv0, sv1, first

            z16 = jnp.zeros((LANES,), jnp.int32)
            sv0, sv1, first = lax.fori_loop(0, BS, seg_a,
                                            (z16, z16, jnp.int32(-1)))
            sv0, sv1 = fix_lane_b(sv0, sv1, first)
            selidx[pl.ds(0, LANES)] = sv0
            selidx[pl.ds(LANES, LANES)] = sv1
            selhist[it + 1, pl.ds(0, LANES)] = sv0
            selhist[it + 1, pl.ds(LANES, LANES)] = sv1
            if it < NSEL - 2:
                copy = pltpu.async_copy(g_hbm.at[selidx], gbuf, sem)

        fill = jnp.full((LANES,), NEG_FILL, jnp.float32)
        for c in range(LOGN // LANES):
            logits_v[pl.ds(c * LANES, LANES)] = fill

        for it in range(NSEL):
            for h in range(2):
                nvec = iota + h * LANES
                live = nvec < BS - 1
                avec = jnp.minimum(nvec + (nvec >= b).astype(jnp.int32),
                                   jnp.int32(BS - 1))
                rows = plsc.load_gather(
                    selhist, [jnp.full((LANES,), it, jnp.int32), avec],
                    mask=live)
                rows = jnp.where(live, rows, 0)
                vals = plsc.load_gather(sim_v, [rows], mask=live)
                posn = jnp.where(live, 1 + nvec * NSEL + it, 0)
                plsc.store_scatter(logits_v, [posn], vals, mask=live)

        p0 = sim_v[pl.ds(b * QN, LANES)]
        p1 = sim_v[pl.ds(b * QN + LANES, LANES)]
        pos = jnp.maximum(jnp.max(p0), jnp.max(p1))
        l0 = logits_v[pl.ds(0, LANES)]
        logits_v[pl.ds(0, LANES)] = jnp.where(iota == 0, pos, l0)

        pltpu.sync_copy(logits_v, out_hbm.at[b])

    return body(G, sim, diff, sel0)


def _loss_body(lg_ref, out_ref):
    lg = lg_ref[...]
    m = jnp.max(lg, axis=1, keepdims=True)
    s = jnp.sum(jnp.exp(lg - m), axis=1, keepdims=True)
    lse = m + jnp.log(s)
    logp0 = lg[:, 0:1] - lse
    out_ref[...] = jnp.full((1, 1), -jnp.mean(logp0), jnp.float32)


def _loss(logits):
    return pl.pallas_call(
        _loss_body,
        out_shape=jax.ShapeDtypeStruct((1, 1), jnp.float32),
    )(logits)


def kernel(vis_fs, lan_fs):
    V = vis_fs.reshape(NROW, FD)
    L = lan_fs.reshape(BS, FD)
    G, sim, diff, sel0 = _prep(V, L)
    logits = _sc_mine(G, sim, diff, sel0)
    return _loss(logits).reshape(())


# speedup vs baseline: 1.1110x; 1.0298x over previous
import functools

import jax
import jax.numpy as jnp
from jax import lax
from jax.experimental import pallas as pl
from jax.experimental.pallas import tpu as pltpu
from jax.experimental.pallas import tpu_sc as plsc

BS = 32
QN = 32
FD = 256
NROW = BS * QN
NSEL = 4
LANES = 16
LOGN = 128
NEG_FILL = -1e30
BIG = 9999


def _f16_roundtrip(x):
    c = jnp.float32(8193.0)
    m = jnp.float32(0.75)
    y = x * c
    hi = y - (y - x)
    lo = (x + m) - m
    return jnp.where(jnp.abs(x) >= jnp.float32(2.0 ** -14), hi, lo)


def _prep_body(v_ref, l_ref, g_ref, sim_ref, diff_ref, sel0_ref):
    V = v_ref[...]
    L = l_ref[...]
    n2 = jnp.sum(V * V, axis=1, keepdims=True)
    nrm = jnp.maximum(jnp.sqrt(n2), 1e-12)
    Uh = _f16_roundtrip(V / nrm)
    G = lax.dot_general(Uh, Uh, (((1,), (1,)), ((), ())),
                        preferred_element_type=jnp.float32)
    c = jnp.float32(8193.0)
    y = G * c
    g_ref[...] = y - (y - G)
    sim = lax.dot_general(L, V, (((1,), (1,)), ((), ())),
                          preferred_element_type=jnp.float32)
    sim_ref[...] = sim
    sim3 = sim.reshape(BS, BS, QN)
    mn = jnp.min(sim3, axis=2, keepdims=True)
    mx = jnp.max(sim3, axis=2, keepdims=True)
    diff_ref[...] = ((sim3 - mn) / (mx - mn)).reshape(BS, NROW)
    qio = lax.broadcasted_iota(jnp.int32, (BS, BS, QN), 2)
    qmin = jnp.min(jnp.where(sim3 == mx, qio, BIG), axis=2)
    sel0_ref[...] = qmin + lax.broadcasted_iota(jnp.int32, (BS, BS), 1) * QN


def _prep(V, L):
    return pl.pallas_call(
        _prep_body,
        out_shape=[
            jax.ShapeDtypeStruct((NROW, NROW), jnp.float32),
            jax.ShapeDtypeStruct((BS, NROW), jnp.float32),
            jax.ShapeDtypeStruct((BS, NROW), jnp.float32),
            jax.ShapeDtypeStruct((BS, BS), jnp.int32),
        ],
    )(V, L)


def _sc_mine(G, sim, diff, sel0):
    info = plsc.get_sparse_core_info()
    nc = info.num_cores

    mesh = plsc.VectorSubcoreMesh(core_axis_name="c", subcore_axis_name="s")

    @functools.partial(
        pl.kernel,
        mesh=mesh,
        compiler_params=pltpu.CompilerParams(needs_layout_passes=False),
        out_type=jax.ShapeDtypeStruct((BS, LOGN), jnp.float32),
        scratch_types=[
            pltpu.VMEM((NROW,), jnp.float32),
            pltpu.VMEM((NROW,), jnp.float32),
            pltpu.VMEM((NROW,), jnp.float32),
            pltpu.VMEM((BS,), jnp.int32),
            pltpu.VMEM((NSEL, BS), jnp.int32),
            pltpu.VMEM((BS, NROW), jnp.float32),
            pltpu.VMEM((BS, NROW), jnp.float32),
            pltpu.VMEM((LOGN,), jnp.float32),
            pltpu.SemaphoreType.DMA,
        ],
    )
    def body(g_hbm, sim_hbm, diff_hbm, sel0_hbm, out_hbm, sim_v, diff_v,
             uniq_v, selidx, selhist, gbuf_a, gbuf_b, logits_v, sem):
        b = lax.axis_index("s") * nc + lax.axis_index("c")
        iota = lax.iota(jnp.int32, LANES)

        pltpu.sync_copy(sim_hbm.at[b], sim_v)
        pltpu.sync_copy(diff_hbm.at[b], diff_v)
        pltpu.sync_copy(sel0_hbm.at[b], selidx)

        def fix_lane_b(sv0, sv1, repl):
            sv0 = jnp.where(iota == b, repl, sv0)
            sv1 = jnp.where(iota == b - LANES, repl, sv1)
            return sv0, sv1

        dupe = plsc.load_gather(selidx, [iota * 0 + (b + 1) % BS])
        sv0, sv1 = fix_lane_b(selidx[pl.ds(0, LANES)],
                              selidx[pl.ds(LANES, LANES)], dupe)
        selidx[pl.ds(0, LANES)] = sv0
        selidx[pl.ds(LANES, LANES)] = sv1
        selhist[0, pl.ds(0, LANES)] = sv0
        selhist[0, pl.ds(LANES, LANES)] = sv1

        gbufs = (gbuf_a, gbuf_b, gbuf_a)
        pending = [pltpu.async_copy(g_hbm.at[selidx], gbuf_a, sem)]

        for it in range(NSEL - 1):
            gbuf = gbufs[it]
            nxt = gbufs[it + 1] if it < NSEL - 2 else None
            for cp in pending:
                cp.wait()
            pending = []

            def seg_a(a, carry):
                sv0, sv1, first = carry
                base = a * QN
                subs = []
                for half in range(2):
                    sl = pl.ds(base + half * LANES, LANES)
                    ms = [jnp.maximum(gbuf[2 * j, sl], gbuf[2 * j + 1, sl])
                          for j in range(BS // 2)]
                    while len(ms) > 1:
                        ms = [jnp.maximum(ms[2 * j], ms[2 * j + 1])
                              for j in range(len(ms) // 2)]
                    qv = (1.0 - ms[0]) * 0.5
                    u = qv if it == 0 else jnp.minimum(uniq_v[sl], qv)
                    uniq_v[sl] = u
                    subs.append(u * diff_v[sl])
                s0, s1 = subs
                m = jnp.maximum(jnp.max(s0), jnp.max(s1))
                q0 = jnp.min(jnp.where(s0 == m, iota, jnp.int32(BIG)))
                q1 = jnp.min(jnp.where(s1 == m, iota + LANES, jnp.int32(BIG)))
                sel = base + jnp.minimum(q0, q1)
                upd = a != b
                sv0 = jnp.where(jnp.logical_and(iota == a, upd), sel, sv0)
                sv1 = jnp.where(jnp.logical_and(iota == a - LANES, upd),
                                sel, sv1)
                first = jnp.where(jnp.logical_and(upd, first < 0), sel, first)
                return sv0, sv1, first

            z16 = jnp.zeros((LANES,), jnp.int32)
            carry = (z16, z16, jnp.int32(-1))
            last = it == NSEL - 2
            if last:
                sv0, sv1, first = lax.fori_loop(0, BS, seg_a, carry)
            else:
                sv0, sv1, first = lax.fori_loop(0, BS // 2, seg_a, carry)
                sv0 = jnp.where(iota == b, first, sv0)
                selidx[pl.ds(0, LANES)] = sv0
                pending.append(pltpu.async_copy(
                    g_hbm.at[selidx.at[pl.ds(0, LANES)]],
                    nxt.at[pl.ds(0, LANES)], sem))
                sv0, sv1, first = lax.fori_loop(BS // 2, BS, seg_a,
                                                (sv0, sv1, first))
            sv0, sv1 = fix_lane_b(sv0, sv1, first)
            selidx[pl.ds(0, LANES)] = sv0
            selidx[pl.ds(LANES, LANES)] = sv1
            selhist[it + 1, pl.ds(0, LANES)] = sv0
            selhist[it + 1, pl.ds(LANES, LANES)] = sv1
            if not last:
                pending.append(pltpu.async_copy(
                    g_hbm.at[selidx.at[pl.ds(LANES, LANES)]],
                    nxt.at[pl.ds(LANES, LANES)], sem))

        fill = jnp.full((LANES,), NEG_FILL, jnp.float32)
        for c in range(LOGN // LANES):
            logits_v[pl.ds(c * LANES, LANES)] = fill

        for it in range(NSEL):
            for h in range(2):
                nvec = iota + h * LANES
                live = nvec < BS - 1
                avec = jnp.minimum(nvec + (nvec >= b).astype(jnp.int32),
                                   jnp.int32(BS - 1))
                rows = plsc.load_gather(
                    selhist, [jnp.full((LANES,), it, jnp.int32), avec],
                    mask=live)
                rows = jnp.where(live, rows, 0)
                vals = plsc.load_gather(sim_v, [rows], mask=live)
                posn = jnp.where(live, 1 + nvec * NSEL + it, 0)
                plsc.store_scatter(logits_v, [posn], vals, mask=live)

        p0 = sim_v[pl.ds(b * QN, LANES)]
        p1 = sim_v[pl.ds(b * QN + LANES, LANES)]
        pos = jnp.maximum(jnp.max(p0), jnp.max(p1))
        l0 = logits_v[pl.ds(0, LANES)]
        logits_v[pl.ds(0, LANES)] = jnp.where(iota == 0, pos, l0)

        pltpu.sync_copy(logits_v, out_hbm.at[b])

    return body(G, sim, diff, sel0)


def _loss_body(lg_ref, out_ref):
    lg = lg_ref[...]
    m = jnp.max(lg, axis=1, keepdims=True)
    s = jnp.sum(jnp.exp(lg - m), axis=1, keepdims=True)
    lse = m + jnp.log(s)
    logp0 = lg[:, 0:1] - lse
    out_ref[...] = jnp.full((1, 1), -jnp.mean(logp0), jnp.float32)


def _loss(logits):
    return pl.pallas_call(
        _loss_body,
        out_shape=jax.ShapeDtypeStruct((1, 1), jnp.float32),
    )(logits)


def kernel(vis_fs, lan_fs):
    V = vis_fs.reshape(NROW, FD)
    L = lan_fs.reshape(BS, FD)
    G, sim, diff, sel0 = _prep(V, L)
    logits = _sc_mine(G, sim, diff, sel0)
    return _loss(logits).reshape(())


# speedup vs baseline: 1.1435x; 1.0293x over previous
import functools

import jax
import jax.numpy as jnp
from jax import lax
from jax.experimental import pallas as pl
from jax.experimental.pallas import tpu as pltpu
from jax.experimental.pallas import tpu_sc as plsc

BS = 32
QN = 32
FD = 256
NROW = BS * QN
NSEL = 4
LANES = 16
LOGN = 128
NEG_FILL = -1e30
BIG = 9999


def _f16_roundtrip(x):
    c = jnp.float32(8193.0)
    m = jnp.float32(0.75)
    y = x * c
    hi = y - (y - x)
    lo = (x + m) - m
    return jnp.where(jnp.abs(x) >= jnp.float32(2.0 ** -14), hi, lo)


def _prep_body(v_ref, l_ref, g_ref, sim_ref, diff_ref, sel0_ref):
    V = v_ref[...]
    L = l_ref[...]
    n2 = jnp.sum(V * V, axis=1, keepdims=True)
    nrm = jnp.maximum(jnp.sqrt(n2), 1e-12)
    Uh = _f16_roundtrip(V / nrm)
    G = lax.dot_general(Uh, Uh, (((1,), (1,)), ((), ())),
                        preferred_element_type=jnp.float32)
    c = jnp.float32(8193.0)
    y = G * c
    g_ref[...] = y - (y - G)
    sim = lax.dot_general(L, V, (((1,), (1,)), ((), ())),
                          preferred_element_type=jnp.float32)
    sim_ref[...] = sim
    sim3 = sim.reshape(BS, BS, QN)
    mn = jnp.min(sim3, axis=2, keepdims=True)
    mx = jnp.max(sim3, axis=2, keepdims=True)
    diff_ref[...] = ((sim3 - mn) / (mx - mn)).reshape(BS, NROW)
    qio = lax.broadcasted_iota(jnp.int32, (BS, BS, QN), 2)
    qmin = jnp.min(jnp.where(sim3 == mx, qio, BIG), axis=2)
    sel0_ref[...] = qmin + lax.broadcasted_iota(jnp.int32, (BS, BS), 1) * QN


def _prep(V, L):
    return pl.pallas_call(
        _prep_body,
        out_shape=[
            jax.ShapeDtypeStruct((NROW, NROW), jnp.float32),
            jax.ShapeDtypeStruct((BS, NROW), jnp.float32),
            jax.ShapeDtypeStruct((BS, NROW), jnp.float32),
            jax.ShapeDtypeStruct((BS, BS), jnp.int32),
        ],
    )(V, L)


def _sc_mine(G, sim, diff, sel0):
    info = plsc.get_sparse_core_info()
    nc = info.num_cores

    mesh = plsc.VectorSubcoreMesh(core_axis_name="c", subcore_axis_name="s")

    @functools.partial(
        pl.kernel,
        mesh=mesh,
        compiler_params=pltpu.CompilerParams(needs_layout_passes=False),
        out_type=jax.ShapeDtypeStruct((BS, LOGN), jnp.float32),
        scratch_types=[
            pltpu.VMEM((NROW,), jnp.float32),
            pltpu.VMEM((NROW,), jnp.float32),
            pltpu.VMEM((NROW,), jnp.float32),
            pltpu.VMEM((BS,), jnp.int32),
            pltpu.VMEM((NSEL, BS), jnp.int32),
            pltpu.VMEM((BS, NROW), jnp.float32),
            pltpu.VMEM((BS, NROW), jnp.float32),
            pltpu.VMEM((LOGN,), jnp.float32),
            pltpu.SemaphoreType.DMA,
        ],
    )
    def body(g_hbm, sim_hbm, diff_hbm, sel0_hbm, out_hbm, sim_v, diff_v,
             uniq_v, selidx, selhist, gbuf_a, gbuf_b, logits_v, sem):
        b = lax.axis_index("s") * nc + lax.axis_index("c")
        iota = lax.iota(jnp.int32, LANES)

        pltpu.sync_copy(sel0_hbm.at[b], selidx)

        def fix_lane_b(sv0, sv1, repl):
            sv0 = jnp.where(iota == b, repl, sv0)
            sv1 = jnp.where(iota == b - LANES, repl, sv1)
            return sv0, sv1

        dupe = plsc.load_gather(selidx, [iota * 0 + (b + 1) % BS])
        sv0, sv1 = fix_lane_b(selidx[pl.ds(0, LANES)],
                              selidx[pl.ds(LANES, LANES)], dupe)
        selidx[pl.ds(0, LANES)] = sv0
        selidx[pl.ds(LANES, LANES)] = sv1
        selhist[0, pl.ds(0, LANES)] = sv0
        selhist[0, pl.ds(LANES, LANES)] = sv1

        gbufs = (gbuf_a, gbuf_b, gbuf_a)
        pending = [pltpu.async_copy(g_hbm.at[selidx], gbuf_a, sem)]
        pltpu.sync_copy(sim_hbm.at[b], sim_v)
        pltpu.sync_copy(diff_hbm.at[b], diff_v)

        for it in range(NSEL - 1):
            gbuf = gbufs[it]
            nxt = gbufs[it + 1] if it < NSEL - 2 else None
            for cp in pending:
                cp.wait()
            pending = []

            def seg_a(a, carry):
                sv0, sv1, first = carry
                base = a * QN
                subs = []
                for half in range(2):
                    sl = pl.ds(base + half * LANES, LANES)
                    ms = [jnp.maximum(gbuf[2 * j, sl], gbuf[2 * j + 1, sl])
                          for j in range(BS // 2)]
                    while len(ms) > 1:
                        ms = [jnp.maximum(ms[2 * j], ms[2 * j + 1])
                              for j in range(len(ms) // 2)]
                    qv = (1.0 - ms[0]) * 0.5
                    u = qv if it == 0 else jnp.minimum(uniq_v[sl], qv)
                    uniq_v[sl] = u
                    subs.append(u * diff_v[sl])
                s0, s1 = subs
                m = jnp.maximum(jnp.max(s0), jnp.max(s1))
                q0 = jnp.min(jnp.where(s0 == m, iota, jnp.int32(BIG)))
                q1 = jnp.min(jnp.where(s1 == m, iota + LANES, jnp.int32(BIG)))
                sel = base + jnp.minimum(q0, q1)
                upd = a != b
                sv0 = jnp.where(jnp.logical_and(iota == a, upd), sel, sv0)
                sv1 = jnp.where(jnp.logical_and(iota == a - LANES, upd),
                                sel, sv1)
                first = jnp.where(jnp.logical_and(upd, first < 0), sel, first)
                return sv0, sv1, first

            z16 = jnp.zeros((LANES,), jnp.int32)
            carry = (z16, z16, jnp.int32(-1))
            last = it == NSEL - 2
            if last:
                sv0, sv1, first = lax.fori_loop(0, BS, seg_a, carry)
            else:
                sv0, sv1, first = lax.fori_loop(0, BS // 2, seg_a, carry)
                sv0 = jnp.where(iota == b, first, sv0)
                selidx[pl.ds(0, LANES)] = sv0
                pending.append(pltpu.async_copy(
                    g_hbm.at[selidx.at[pl.ds(0, LANES)]],
                    nxt.at[pl.ds(0, LANES)], sem))
                sv0, sv1, first = lax.fori_loop(BS // 2, BS, seg_a,
                                                (sv0, sv1, first))
            sv0, sv1 = fix_lane_b(sv0, sv1, first)
            selidx[pl.ds(0, LANES)] = sv0
            selidx[pl.ds(LANES, LANES)] = sv1
            selhist[it + 1, pl.ds(0, LANES)] = sv0
            selhist[it + 1, pl.ds(LANES, LANES)] = sv1
            if not last:
                pending.append(pltpu.async_copy(
                    g_hbm.at[selidx.at[pl.ds(LANES, LANES)]],
                    nxt.at[pl.ds(LANES, LANES)], sem))

        fill = jnp.full((LANES,), NEG_FILL, jnp.float32)
        for c in range(LOGN // LANES):
            logits_v[pl.ds(c * LANES, LANES)] = fill

        for it in range(NSEL):
            for h in range(2):
                nvec = iota + h * LANES
                live = nvec < BS - 1
                avec = jnp.minimum(nvec + (nvec >= b).astype(jnp.int32),
                                   jnp.int32(BS - 1))
                rows = plsc.load_gather(
                    selhist, [jnp.full((LANES,), it, jnp.int32), avec],
                    mask=live)
                rows = jnp.where(live, rows, 0)
                vals = plsc.load_gather(sim_v, [rows], mask=live)
                posn = jnp.where(live, 1 + nvec * NSEL + it, 0)
                plsc.store_scatter(logits_v, [posn], vals, mask=live)

        p0 = sim_v[pl.ds(b * QN, LANES)]
        p1 = sim_v[pl.ds(b * QN + LANES, LANES)]
        pos = jnp.maximum(jnp.max(p0), jnp.max(p1))
        l0 = logits_v[pl.ds(0, LANES)]
        logits_v[pl.ds(0, LANES)] = jnp.where(iota == 0, pos, l0)

        pltpu.sync_copy(logits_v, out_hbm.at[b])

    return body(G, sim, diff, sel0)


def _loss_body(lg_ref, out_ref):
    lg = lg_ref[...]
    m = jnp.max(lg, axis=1, keepdims=True)
    s = jnp.sum(jnp.exp(lg - m), axis=1, keepdims=True)
    lse = m + jnp.log(s)
    logp0 = lg[:, 0:1] - lse
    out_ref[...] = jnp.full((1, 1), -jnp.mean(logp0), jnp.float32)


def _loss(logits):
    return pl.pallas_call(
        _loss_body,
        out_shape=jax.ShapeDtypeStruct((1, 1), jnp.float32),
    )(logits)


def kernel(vis_fs, lan_fs):
    V = vis_fs.reshape(NROW, FD)
    L = lan_fs.reshape(BS, FD)
    G, sim, diff, sel0 = _prep(V, L)
    logits = _sc_mine(G, sim, diff, sel0)
    return _loss(logits).reshape(())
